# Initial kernel scaffold; baseline (speedup 1.0000x reference)
import jax
import jax.numpy as jnp
from jax.experimental import pallas as pl


def kernel(inputs, edge_index, W1, W2, attn_l, attn_r):
    raise NotImplementedError("write your pallas kernel here")


# speedup vs baseline: 10.0018x; 10.0018x over previous
import functools

import jax
import jax.numpy as jnp
from jax import lax
from jax.experimental import pallas as pl
from jax.experimental.pallas import tpu as pltpu
from jax.experimental.pallas import tpu_sc as plsc

N_NODES = 10000
NPAD = 10240
N_EDGES = 320000
IN_DIM = 128
OUT_DIM = 16
NUM_HEADS = 8
ALPHA = 0.2

HH = NUM_HEADS // 2
FH = HH * OUT_DIM
ACCW = IN_DIM
CH = 32
NSUB = 16
E_PAD = 320512
EPT = E_PAD // NSUB
NFULL = EPT // CH
NHALF = NPAD // 2
SROWS = 5632
RPT = SROWS // NSUB
RBLK = CH
NBLK = RPT // RBLK


def _dense_body(x_ref, w1_ref, w2_ref, am_ref, ft_ref, aa_ref):
    x = x_ref[...]
    ft1 = lax.dot_general(x, w1_ref[...], (((1,), (1,)), ((), ())),
                          preferred_element_type=jnp.float32)
    h2 = jnp.maximum(ft1, ALPHA * ft1)
    ft2 = lax.dot_general(h2, w2_ref[...], (((1,), (1,)), ((), ())),
                          preferred_element_type=jnp.float32)
    ft_ref[...] = ft2
    aa_ref[...] = jnp.dot(ft2, am_ref[...], preferred_element_type=jnp.float32)


def _dense(x, w1, w2, am):
    blk = 1024
    grid = NPAD // blk
    return pl.pallas_call(
        _dense_body,
        grid=(grid,),
        in_specs=[
            pl.BlockSpec((blk, IN_DIM), lambda i: (i, 0)),
            pl.BlockSpec((IN_DIM, IN_DIM), lambda i: (0, 0)),
            pl.BlockSpec((IN_DIM, IN_DIM), lambda i: (0, 0)),
            pl.BlockSpec((IN_DIM, 2 * NUM_HEADS), lambda i: (0, 0)),
        ],
        out_specs=[
            pl.BlockSpec((blk, IN_DIM), lambda i: (i, 0)),
            pl.BlockSpec((blk, 2 * NUM_HEADS), lambda i: (i, 0)),
        ],
        out_shape=[
            jax.ShapeDtypeStruct((NPAD, IN_DIM), jnp.float32),
            jax.ShapeDtypeStruct((NPAD, 2 * NUM_HEADS), jnp.float32),
        ],
    )(x, w1, w2, am)


def _edge_body(tbl0, tbl1, src_r, dstg_r, d0_r, d1_r, out0, out1,
               sv, dv, dsc, gbuf, hbuf, abuf, sacc, sem, sema, semb):
    c = lax.axis_index("c")

    @pl.when(c == 0)
    def _():
        _core_body(tbl0, out0, src_r, dstg_r, d0_r, d1_r,
                   sv, dv, dsc, gbuf, hbuf, abuf, sacc, sem, sema, semb)

    @pl.when(c == 1)
    def _():
        _core_body(tbl1, out1, src_r, dstg_r, d0_r, d1_r,
                   sv, dv, dsc, gbuf, hbuf, abuf, sacc, sem, sema, semb)


def _core_body(tbl, out, src_r, dstg_r, d0_r, d1_r,
               sv, dv, dsc, gbuf, hbuf, abuf, sacc, sem, sema, semb):
    s = lax.axis_index("s")
    r0 = s * RPT
    iota16 = lax.iota(jnp.int32, 16)
    rq = iota16 >> 2
    hm = iota16 & 3
    ebase = s * EPT
    zero = jnp.zeros((16,), jnp.float32)

    for rnd, dr in ((0, d0_r), (1, d1_r)):
        def zrow(r, carry):
            for j in range(ACCW // 16):
                abuf[r, pl.ds(j * 16, 16)] = zero
            return carry

        lax.fori_loop(0, CH, zrow, 0)
        for b in range(NBLK):
            pltpu.sync_copy(abuf.at[pl.ds(0, RBLK)],
                            sacc.at[pl.ds(r0 + b * RBLK, RBLK)])

        plsc.subcore_barrier()

        def chunk_loop(k, carry):
            base = ebase + k * CH
            pltpu.sync_copy(src_r.at[pl.ds(base, CH)], sv)
            pltpu.sync_copy(dstg_r.at[pl.ds(base, CH)], dv)
            pltpu.sync_copy(dr.at[pl.ds(base, CH)], dsc)
            gcp = pltpu.async_copy(tbl.at[sv], gbuf, sem)
            hcp = pltpu.async_copy(tbl.at[dv], hbuf, sema)
            gcp.wait()
            hcp.wait()

            def wgrp(g, carry):
                rows = g * 4 + rq
                va = plsc.load_gather(gbuf, [rows, FH + hm])
                vb = plsc.load_gather(hbuf, [rows, FH + HH + hm])
                sab = va + vb
                w = jnp.exp(jnp.maximum(sab, ALPHA * sab))
                plsc.store_scatter(abuf, [rows, FH + hm], w)
                return carry

            lax.fori_loop(0, CH // 4, wgrp, 0)

            def scale(i, carry):
                e = i >> 2
                h = i & 3
                mult = plsc.load_gather(
                    abuf, [jnp.full((16,), e, jnp.int32),
                           jnp.full((16,), FH + h, jnp.int32)])
                abuf[e, pl.ds(h * 16, 16)] = gbuf[e, pl.ds(h * 16, 16)] * mult
                return carry

            lax.fori_loop(0, HH * CH, scale, 0)
            pltpu.sync_copy(abuf, sacc.at[dsc], add=True)
            return carry

        lax.fori_loop(0, NFULL, chunk_loop, 0)

        plsc.subcore_barrier()

        for b in range(NBLK):
            rb = r0 + b * RBLK

            @pl.when(rb < NHALF)
            def _():
                pltpu.sync_copy(sacc.at[pl.ds(rb, RBLK)],
                                abuf.at[pl.ds(0, RBLK)])

                def divrow(i, carry):
                    e = i >> 2
                    h = i & 3
                    den = plsc.load_gather(
                        abuf, [jnp.full((16,), e, jnp.int32),
                               jnp.full((16,), FH + h, jnp.int32)])
                    den = jnp.maximum(den, 1e-30)
                    gbuf[e, pl.ds(h * 16, 16)] = abuf[e, pl.ds(h * 16, 16)] / den
                    return carry

                lax.fori_loop(0, HH * RBLK, divrow, 0)
                pltpu.sync_copy(gbuf, out.at[pl.ds(rnd * NHALF + rb, RBLK)])

        plsc.subcore_barrier()


_edge_kernel = functools.partial(
    pl.kernel,
    out_type=(
        jax.ShapeDtypeStruct((NPAD, IN_DIM), jnp.float32),
        jax.ShapeDtypeStruct((NPAD, IN_DIM), jnp.float32),
    ),
    mesh=plsc.VectorSubcoreMesh(core_axis_name="c", subcore_axis_name="s"),
    compiler_params=pltpu.CompilerParams(needs_layout_passes=False),
    scratch_types=[
        pltpu.VMEM((CH,), jnp.int32),
        pltpu.VMEM((CH,), jnp.int32),
        pltpu.VMEM((CH,), jnp.int32),
        pltpu.VMEM((CH, IN_DIM), jnp.float32),
        pltpu.VMEM((CH, IN_DIM), jnp.float32),
        pltpu.VMEM((CH, ACCW), jnp.float32),
        pltpu.VMEM_SHARED((SROWS, ACCW), jnp.float32),
        pltpu.SemaphoreType.DMA,
        pltpu.SemaphoreType.DMA,
        pltpu.SemaphoreType.DMA,
    ],
)(_edge_body)


def kernel(inputs, edge_index, W1, W2, attn_l, attn_r):
    eye = jnp.eye(NUM_HEADS, dtype=jnp.float32)
    ml = (attn_l[:, :, 0][:, :, None] * eye[:, None, :]).reshape(
        NUM_HEADS * OUT_DIM, NUM_HEADS)
    mr = (attn_r[:, :, 0][:, :, None] * eye[:, None, :]).reshape(
        NUM_HEADS * OUT_DIM, NUM_HEADS)
    am = jnp.concatenate([ml, mr], axis=1)

    xpad = jnp.pad(inputs, ((0, NPAD - N_NODES), (0, 0)))
    ft, aa = _dense(xpad, W1, W2, am)
    zpad = jnp.zeros((NPAD, IN_DIM - FH - 2 * HH), jnp.float32)
    tbl0 = jnp.concatenate(
        [ft[:, :FH], aa[:, 0:HH], aa[:, NUM_HEADS:NUM_HEADS + HH], zpad], axis=1)
    tbl1 = jnp.concatenate(
        [ft[:, FH:], aa[:, HH:NUM_HEADS], aa[:, NUM_HEADS + HH:], zpad], axis=1)

    pad = jnp.full((E_PAD - N_EDGES,), NPAD - 1, jnp.int32)
    src = jnp.concatenate([edge_index[0].astype(jnp.int32), pad])
    dst = jnp.concatenate([edge_index[1].astype(jnp.int32), pad])
    dump = NHALF + (dst & 7)
    d0 = jnp.where(dst < NHALF, dst, dump)
    d1 = jnp.where(dst >= NHALF, dst - NHALF, dump)

    out0, out1 = _edge_kernel(tbl0, tbl1, src, dst, d0, d1)
    return jnp.concatenate(
        [out0[:N_NODES, :FH].reshape(N_NODES, HH, OUT_DIM),
         out1[:N_NODES, :FH].reshape(N_NODES, HH, OUT_DIM)], axis=1)


# speedup vs baseline: 15.8270x; 1.5824x over previous
import functools

import jax
import jax.numpy as jnp
from jax import lax
from jax.experimental import pallas as pl
from jax.experimental.pallas import tpu as pltpu
from jax.experimental.pallas import tpu_sc as plsc

N_NODES = 10000
NPAD = 10240
N_EDGES = 320000
IN_DIM = 128
OUT_DIM = 16
NUM_HEADS = 8
ALPHA = 0.2

HH = NUM_HEADS // 2
FH = HH * OUT_DIM
ACCW = IN_DIM
CH = 64
CHH = CH // 2
NSUB = 16
E_PAD = 320512
EPT = E_PAD // NSUB
NFULL = EPT // CH
NHALF = NPAD // 2
SROWS = 5632
RPT = SROWS // NSUB
RBLK = CHH
NBLK = RPT // RBLK


def _dense_body(x_ref, w1_ref, w2_ref, am_ref, ft_ref, aa_ref):
    x = x_ref[...]
    ft1 = lax.dot_general(x, w1_ref[...], (((1,), (1,)), ((), ())),
                          preferred_element_type=jnp.float32)
    h2 = jnp.maximum(ft1, ALPHA * ft1)
    ft2 = lax.dot_general(h2, w2_ref[...], (((1,), (1,)), ((), ())),
                          preferred_element_type=jnp.float32)
    ft_ref[...] = ft2
    aa_ref[...] = jnp.dot(ft2, am_ref[...], preferred_element_type=jnp.float32)


def _dense(x, w1, w2, am):
    blk = 1024
    grid = NPAD // blk
    return pl.pallas_call(
        _dense_body,
        grid=(grid,),
        in_specs=[
            pl.BlockSpec((blk, IN_DIM), lambda i: (i, 0)),
            pl.BlockSpec((IN_DIM, IN_DIM), lambda i: (0, 0)),
            pl.BlockSpec((IN_DIM, IN_DIM), lambda i: (0, 0)),
            pl.BlockSpec((IN_DIM, 2 * NUM_HEADS), lambda i: (0, 0)),
        ],
        out_specs=[
            pl.BlockSpec((blk, IN_DIM), lambda i: (i, 0)),
            pl.BlockSpec((blk, 2 * NUM_HEADS), lambda i: (i, 0)),
        ],
        out_shape=[
            jax.ShapeDtypeStruct((NPAD, IN_DIM), jnp.float32),
            jax.ShapeDtypeStruct((NPAD, 2 * NUM_HEADS), jnp.float32),
        ],
    )(x, w1, w2, am)


def _edge_body(tbl0, tbl1, src_r, dstg_r, d0_r, d1_r, out0, out1,
               sv, dv, dsca, dscb, gbuf, hbuf, abuf, sacc,
               sem, sema, semb, semc):
    c = lax.axis_index("c")

    @pl.when(c == 0)
    def _():
        _core_body(tbl0, out0, src_r, dstg_r, d0_r, d1_r,
                   sv, dv, dsca, dscb, gbuf, hbuf, abuf, sacc,
                   sem, sema, semb, semc)

    @pl.when(c == 1)
    def _():
        _core_body(tbl1, out1, src_r, dstg_r, d0_r, d1_r,
                   sv, dv, dsca, dscb, gbuf, hbuf, abuf, sacc,
                   sem, sema, semb, semc)


def _core_body(tbl, out, src_r, dstg_r, d0_r, d1_r,
               sv, dv, dsca, dscb, gbuf, hbuf, abuf, sacc,
               sem, sema, semb, semc):
    s = lax.axis_index("s")
    r0 = s * RPT
    iota16 = lax.iota(jnp.int32, 16)
    rq = iota16 >> 2
    hm = iota16 & 3
    ebase = s * EPT
    zero = jnp.zeros((16,), jnp.float32)

    for rnd, dr in ((0, d0_r), (1, d1_r)):
        def zrow(r, carry):
            for j in range(ACCW // 16):
                abuf[r, pl.ds(j * 16, 16)] = zero
            return carry

        lax.fori_loop(0, CHH, zrow, 0)
        for b in range(NBLK):
            pltpu.sync_copy(abuf.at[pl.ds(0, RBLK)],
                            sacc.at[pl.ds(r0 + b * RBLK, RBLK)])

        plsc.subcore_barrier()

        def chunk_loop(k, carry):
            base = ebase + k * CH
            c1 = pltpu.async_copy(src_r.at[pl.ds(base, CH)], sv, sem)
            c2 = pltpu.async_copy(dstg_r.at[pl.ds(base, CH)], dv, sema)
            c3 = pltpu.async_copy(dr.at[pl.ds(base, CHH)], dsca, semb)
            c4 = pltpu.async_copy(dr.at[pl.ds(base + CHH, CHH)], dscb, semc)
            c1.wait()
            c2.wait()
            c3.wait()
            c4.wait()
            gcp = pltpu.async_copy(tbl.at[sv], gbuf, sem)
            hcp = pltpu.async_copy(tbl.at[dv], hbuf, sema)
            gcp.wait()
            hcp.wait()

            for half, dsc_ in ((0, dsca), (1, dscb)):
                roff = half * CHH

                def wgrp(g, carry):
                    rows = g * 4 + rq
                    va = plsc.load_gather(gbuf, [roff + rows, FH + hm])
                    vb = plsc.load_gather(hbuf, [roff + rows, FH + HH + hm])
                    sab = va + vb
                    w = jnp.exp(jnp.maximum(sab, ALPHA * sab))
                    plsc.store_scatter(abuf, [rows, FH + hm], w)
                    return carry

                lax.fori_loop(0, CHH // 4, wgrp, 0)

                def scale(i, carry):
                    e = i >> 2
                    h = i & 3
                    mult = plsc.load_gather(
                        abuf, [jnp.full((16,), e, jnp.int32),
                               jnp.full((16,), FH + h, jnp.int32)])
                    abuf[e, pl.ds(h * 16, 16)] = (
                        gbuf[roff + e, pl.ds(h * 16, 16)] * mult)
                    return carry

                lax.fori_loop(0, HH * CHH, scale, 0)
                pltpu.sync_copy(abuf, sacc.at[dsc_], add=True)
            return carry

        lax.fori_loop(0, NFULL, chunk_loop, 0)

        plsc.subcore_barrier()

        for b in range(NBLK):
            rb = r0 + b * RBLK

            @pl.when(rb < NHALF)
            def _():
                pltpu.sync_copy(sacc.at[pl.ds(rb, RBLK)],
                                abuf.at[pl.ds(0, RBLK)])

                def divrow(i, carry):
                    e = i >> 2
                    h = i & 3
                    den = plsc.load_gather(
                        abuf, [jnp.full((16,), e, jnp.int32),
                               jnp.full((16,), FH + h, jnp.int32)])
                    den = jnp.maximum(den, 1e-30)
                    gbuf[e, pl.ds(h * 16, 16)] = abuf[e, pl.ds(h * 16, 16)] / den
                    return carry

                lax.fori_loop(0, HH * RBLK, divrow, 0)
                pltpu.sync_copy(gbuf.at[pl.ds(0, RBLK)],
                                out.at[pl.ds(rnd * NHALF + rb, RBLK)])

        plsc.subcore_barrier()


_edge_kernel = functools.partial(
    pl.kernel,
    out_type=(
        jax.ShapeDtypeStruct((NPAD, IN_DIM), jnp.float32),
        jax.ShapeDtypeStruct((NPAD, IN_DIM), jnp.float32),
    ),
    mesh=plsc.VectorSubcoreMesh(core_axis_name="c", subcore_axis_name="s"),
    compiler_params=pltpu.CompilerParams(needs_layout_passes=False),
    scratch_types=[
        pltpu.VMEM((CH,), jnp.int32),
        pltpu.VMEM((CH,), jnp.int32),
        pltpu.VMEM((CHH,), jnp.int32),
        pltpu.VMEM((CHH,), jnp.int32),
        pltpu.VMEM((CH, IN_DIM), jnp.float32),
        pltpu.VMEM((CH, IN_DIM), jnp.float32),
        pltpu.VMEM((CHH, ACCW), jnp.float32),
        pltpu.VMEM_SHARED((SROWS, ACCW), jnp.float32),
        pltpu.SemaphoreType.DMA,
        pltpu.SemaphoreType.DMA,
        pltpu.SemaphoreType.DMA,
        pltpu.SemaphoreType.DMA,
    ],
)(_edge_body)


def kernel(inputs, edge_index, W1, W2, attn_l, attn_r):
    eye = jnp.eye(NUM_HEADS, dtype=jnp.float32)
    ml = (attn_l[:, :, 0][:, :, None] * eye[:, None, :]).reshape(
        NUM_HEADS * OUT_DIM, NUM_HEADS)
    mr = (attn_r[:, :, 0][:, :, None] * eye[:, None, :]).reshape(
        NUM_HEADS * OUT_DIM, NUM_HEADS)
    am = jnp.concatenate([ml, mr], axis=1)

    xpad = jnp.pad(inputs, ((0, NPAD - N_NODES), (0, 0)))
    ft, aa = _dense(xpad, W1, W2, am)
    zpad = jnp.zeros((NPAD, IN_DIM - FH - 2 * HH), jnp.float32)
    tbl0 = jnp.concatenate(
        [ft[:, :FH], aa[:, 0:HH], aa[:, NUM_HEADS:NUM_HEADS + HH], zpad], axis=1)
    tbl1 = jnp.concatenate(
        [ft[:, FH:], aa[:, HH:NUM_HEADS], aa[:, NUM_HEADS + HH:], zpad], axis=1)

    pad = jnp.full((E_PAD - N_EDGES,), NPAD - 1, jnp.int32)
    src = jnp.concatenate([edge_index[0].astype(jnp.int32), pad])
    dst = jnp.concatenate([edge_index[1].astype(jnp.int32), pad])
    dump = NHALF + (dst & 7)
    d0 = jnp.where(dst < NHALF, dst, dump)
    d1 = jnp.where(dst >= NHALF, dst - NHALF, dump)

    out0, out1 = _edge_kernel(tbl0, tbl1, src, dst, d0, d1)
    return jnp.concatenate(
        [out0[:N_NODES, :FH].reshape(N_NODES, HH, OUT_DIM),
         out1[:N_NODES, :FH].reshape(N_NODES, HH, OUT_DIM)], axis=1)


# speedup vs baseline: 17.7187x; 1.1195x over previous
import functools

import jax
import jax.numpy as jnp
from jax import lax
from jax.experimental import pallas as pl
from jax.experimental.pallas import tpu as pltpu
from jax.experimental.pallas import tpu_sc as plsc

N_NODES = 10000
NPAD = 10240
N_EDGES = 320000
IN_DIM = 128
OUT_DIM = 16
NUM_HEADS = 8
ALPHA = 0.2

HH = NUM_HEADS // 2
FH = HH * OUT_DIM
ACCW = IN_DIM
CH = 64
CHH = CH // 2
NSUB = 16
E_PAD = 320512
EPT = E_PAD // NSUB
NFULL = EPT // CH
NHALF = NPAD // 2
SROWS = 5632
RPT = SROWS // NSUB
RBLK = CHH
NBLK = RPT // RBLK


def _dense_body(x_ref, w1_ref, w2_ref, am_ref, ft_ref, aa_ref):
    x = x_ref[...]
    ft1 = lax.dot_general(x, w1_ref[...], (((1,), (1,)), ((), ())),
                          preferred_element_type=jnp.float32)
    h2 = jnp.maximum(ft1, ALPHA * ft1)
    ft2 = lax.dot_general(h2, w2_ref[...], (((1,), (1,)), ((), ())),
                          preferred_element_type=jnp.float32)
    ft_ref[...] = ft2
    aa_ref[...] = jnp.dot(ft2, am_ref[...], preferred_element_type=jnp.float32)


def _dense(x, w1, w2, am):
    blk = 1024
    grid = NPAD // blk
    return pl.pallas_call(
        _dense_body,
        grid=(grid,),
        in_specs=[
            pl.BlockSpec((blk, IN_DIM), lambda i: (i, 0)),
            pl.BlockSpec((IN_DIM, IN_DIM), lambda i: (0, 0)),
            pl.BlockSpec((IN_DIM, IN_DIM), lambda i: (0, 0)),
            pl.BlockSpec((IN_DIM, 2 * NUM_HEADS), lambda i: (0, 0)),
        ],
        out_specs=[
            pl.BlockSpec((blk, IN_DIM), lambda i: (i, 0)),
            pl.BlockSpec((blk, 2 * NUM_HEADS), lambda i: (i, 0)),
        ],
        out_shape=[
            jax.ShapeDtypeStruct((NPAD, IN_DIM), jnp.float32),
            jax.ShapeDtypeStruct((NPAD, 2 * NUM_HEADS), jnp.float32),
        ],
    )(x, w1, w2, am)


def _edge_body(tbl0, tbl1, src_r, dstg_r, d0_r, d1_r, out0, out1,
               svA, dvA, daA, dbA, svB, dvB, daB, dbB,
               gbuf, hbuf, abuf, sacc, *sems):
    c = lax.axis_index("c")

    @pl.when(c == 0)
    def _():
        _core_body(tbl0, out0, src_r, dstg_r, d0_r, d1_r,
                   svA, dvA, daA, dbA, svB, dvB, daB, dbB,
                   gbuf, hbuf, abuf, sacc, sems)

    @pl.when(c == 1)
    def _():
        _core_body(tbl1, out1, src_r, dstg_r, d0_r, d1_r,
                   svA, dvA, daA, dbA, svB, dvB, daB, dbB,
                   gbuf, hbuf, abuf, sacc, sems)


def _core_body(tbl, out, src_r, dstg_r, d0_r, d1_r,
               svA, dvA, daA, dbA, svB, dvB, daB, dbB,
               gbuf, hbuf, abuf, sacc, sems):
    bufsA = (svA, dvA, daA, dbA)
    bufsB = (svB, dvB, daB, dbB)
    semsA = sems[0:4]
    semsB = sems[4:8]
    sg, sh = sems[8], sems[9]
    s = lax.axis_index("s")
    r0 = s * RPT
    iota16 = lax.iota(jnp.int32, 16)
    rq = iota16 >> 2
    hm = iota16 & 3
    ebase = s * EPT
    zero = jnp.zeros((16,), jnp.float32)

    for rnd, dr in ((0, d0_r), (1, d1_r)):
        def zrow(r, carry):
            for j in range(ACCW // 16):
                abuf[r, pl.ds(j * 16, 16)] = zero
            return carry

        lax.fori_loop(0, CHH, zrow, 0)
        for b in range(NBLK):
            pltpu.sync_copy(abuf.at[pl.ds(0, RBLK)],
                            sacc.at[pl.ds(r0 + b * RBLK, RBLK)])

        plsc.subcore_barrier()

        def issue_idx(k, bufs, ss):
            base = ebase + k * CH
            pltpu.async_copy(src_r.at[pl.ds(base, CH)], bufs[0], ss[0])
            pltpu.async_copy(dstg_r.at[pl.ds(base, CH)], bufs[1], ss[1])
            pltpu.async_copy(dr.at[pl.ds(base, CHH)], bufs[2], ss[2])
            pltpu.async_copy(dr.at[pl.ds(base + CHH, CHH)], bufs[3], ss[3])

        def wait_idx(bufs, ss):
            pltpu.make_async_copy(src_r.at[pl.ds(0, CH)], bufs[0], ss[0]).wait()
            pltpu.make_async_copy(dstg_r.at[pl.ds(0, CH)], bufs[1], ss[1]).wait()
            pltpu.make_async_copy(dr.at[pl.ds(0, CHH)], bufs[2], ss[2]).wait()
            pltpu.make_async_copy(dr.at[pl.ds(0, CHH)], bufs[3], ss[3]).wait()

        def compute_chunk(bufs):
            sv_, dv_, da_, db_ = bufs
            gcp = pltpu.async_copy(tbl.at[sv_], gbuf, sg)
            hcp = pltpu.async_copy(tbl.at[dv_], hbuf, sh)
            gcp.wait()
            hcp.wait()

            for half, dsc_ in ((0, da_), (1, db_)):
                roff = half * CHH

                def wgrp(g, carry):
                    rows = g * 4 + rq
                    va = plsc.load_gather(gbuf, [roff + rows, FH + hm])
                    vb = plsc.load_gather(hbuf, [roff + rows, FH + HH + hm])
                    sab = va + vb
                    w = jnp.exp(jnp.maximum(sab, ALPHA * sab))
                    plsc.store_scatter(abuf, [rows, FH + hm], w)
                    return carry

                lax.fori_loop(0, CHH // 4, wgrp, 0)

                def scale(i, carry):
                    e = i >> 2
                    h = i & 3
                    mult = plsc.load_gather(
                        abuf, [jnp.full((16,), e, jnp.int32),
                               jnp.full((16,), FH + h, jnp.int32)])
                    abuf[e, pl.ds(h * 16, 16)] = (
                        gbuf[roff + e, pl.ds(h * 16, 16)] * mult)
                    return carry

                lax.fori_loop(0, HH * CHH, scale, 0)
                pltpu.sync_copy(abuf, sacc.at[dsc_], add=True)

        issue_idx(0, bufsA, semsA)

        def pair_loop(j, carry):
            wait_idx(bufsA, semsA)
            issue_idx(2 * j + 1, bufsB, semsB)
            compute_chunk(bufsA)
            wait_idx(bufsB, semsB)
            issue_idx(2 * j + 2, bufsA, semsA)
            compute_chunk(bufsB)
            return carry

        lax.fori_loop(0, NFULL // 2, pair_loop, 0)
        wait_idx(bufsA, semsA)
        compute_chunk(bufsA)

        plsc.subcore_barrier()

        for b in range(NBLK):
            rb = r0 + b * RBLK

            @pl.when(rb < NHALF)
            def _():
                pltpu.sync_copy(sacc.at[pl.ds(rb, RBLK)],
                                abuf.at[pl.ds(0, RBLK)])

                def divrow(i, carry):
                    e = i >> 2
                    h = i & 3
                    den = plsc.load_gather(
                        abuf, [jnp.full((16,), e, jnp.int32),
                               jnp.full((16,), FH + h, jnp.int32)])
                    den = jnp.maximum(den, 1e-30)
                    gbuf[e, pl.ds(h * 16, 16)] = abuf[e, pl.ds(h * 16, 16)] / den
                    return carry

                lax.fori_loop(0, HH * RBLK, divrow, 0)
                pltpu.sync_copy(gbuf.at[pl.ds(0, RBLK)],
                                out.at[pl.ds(rnd * NHALF + rb, RBLK)])

        plsc.subcore_barrier()


_edge_kernel = functools.partial(
    pl.kernel,
    out_type=(
        jax.ShapeDtypeStruct((NPAD, IN_DIM), jnp.float32),
        jax.ShapeDtypeStruct((NPAD, IN_DIM), jnp.float32),
    ),
    mesh=plsc.VectorSubcoreMesh(core_axis_name="c", subcore_axis_name="s"),
    compiler_params=pltpu.CompilerParams(needs_layout_passes=False),
    scratch_types=[
        pltpu.VMEM((CH,), jnp.int32),
        pltpu.VMEM((CH,), jnp.int32),
        pltpu.VMEM((CHH,), jnp.int32),
        pltpu.VMEM((CHH,), jnp.int32),
        pltpu.VMEM((CH,), jnp.int32),
        pltpu.VMEM((CH,), jnp.int32),
        pltpu.VMEM((CHH,), jnp.int32),
        pltpu.VMEM((CHH,), jnp.int32),
        pltpu.VMEM((CH, IN_DIM), jnp.float32),
        pltpu.VMEM((CH, IN_DIM), jnp.float32),
        pltpu.VMEM((CHH, ACCW), jnp.float32),
        pltpu.VMEM_SHARED((SROWS, ACCW), jnp.float32),
    ] + [pltpu.SemaphoreType.DMA] * 10,
)(_edge_body)


def kernel(inputs, edge_index, W1, W2, attn_l, attn_r):
    eye = jnp.eye(NUM_HEADS, dtype=jnp.float32)
    ml = (attn_l[:, :, 0][:, :, None] * eye[:, None, :]).reshape(
        NUM_HEADS * OUT_DIM, NUM_HEADS)
    mr = (attn_r[:, :, 0][:, :, None] * eye[:, None, :]).reshape(
        NUM_HEADS * OUT_DIM, NUM_HEADS)
    am = jnp.concatenate([ml, mr], axis=1)

    xpad = jnp.pad(inputs, ((0, NPAD - N_NODES), (0, 0)))
    ft, aa = _dense(xpad, W1, W2, am)
    zpad = jnp.zeros((NPAD, IN_DIM - FH - 2 * HH), jnp.float32)
    tbl0 = jnp.concatenate(
        [ft[:, :FH], aa[:, 0:HH], aa[:, NUM_HEADS:NUM_HEADS + HH], zpad], axis=1)
    tbl1 = jnp.concatenate(
        [ft[:, FH:], aa[:, HH:NUM_HEADS], aa[:, NUM_HEADS + HH:], zpad], axis=1)

    pad = jnp.full((E_PAD - N_EDGES,), NPAD - 1, jnp.int32)
    src = jnp.concatenate([edge_index[0].astype(jnp.int32), pad])
    dst = jnp.concatenate([edge_index[1].astype(jnp.int32), pad])
    dump = NHALF + (dst & 7)
    d0 = jnp.where(dst < NHALF, dst, dump)
    d1 = jnp.where(dst >= NHALF, dst - NHALF, dump)

    out0, out1 = _edge_kernel(tbl0, tbl1, src, dst, d0, d1)
    return jnp.concatenate(
        [out0[:N_NODES, :FH].reshape(N_NODES, HH, OUT_DIM),
         out1[:N_NODES, :FH].reshape(N_NODES, HH, OUT_DIM)], axis=1)
